# baseline (speedup 1.0000x reference)
import jax
import jax.numpy as jnp
from jax.experimental import pallas as pl

N_USERS = 25000
N_ITEMS = 25000
EMB = 64
HYP = 128
K_LAYERS = 2
C_LAYERS = 2
SLOPE = 0.1


def _leaky(x):
    return jnp.where(x >= 0, x, SLOPE * x)


def _add_kernel(a_ref, b_ref, o_ref):
    o_ref[...] = a_ref[...] + b_ref[...]


def _padd(a, b):
    blk = 2000
    return pl.pallas_call(
        _add_kernel,
        out_shape=jax.ShapeDtypeStruct(a.shape, a.dtype),
        grid=(a.shape[0] // blk,),
        in_specs=[
            pl.BlockSpec((blk, a.shape[1]), lambda i: (i, 0)),
            pl.BlockSpec((blk, a.shape[1]), lambda i: (i, 0)),
        ],
        out_specs=pl.BlockSpec((blk, a.shape[1]), lambda i: (i, 0)),
    )(a, b)


def kernel(adj_indices, adj_values, uEmbeds, iEmbeds, uHyper, iHyper, V, keepRate):
    N = N_USERS + N_ITEMS
    embeds = jnp.concatenate([uEmbeds, iEmbeds], axis=0)
    lats = [embeds]
    gnnLats = []
    hyperLats = []
    uuHyper = uEmbeds @ uHyper
    iiHyper = iEmbeds @ iHyper

    def spmm(indices, values, x, n):
        rows = indices[0]
        cols = indices[1]
        msgs = values[:, None] * jnp.take(x, cols, axis=0)
        return jax.ops.segment_sum(msgs, rows, num_segments=n)

    def hgnn(adjd, embeds_h, V):
        lat = _leaky(adjd.T @ embeds_h)
        for _ in range(C_LAYERS):
            lat = _leaky(V @ lat) + lat
        return _leaky(adjd @ lat)

    for _ in range(K_LAYERS):
        temEmbeds = _leaky(spmm(adj_indices, adj_values, lats[-1], N))
        hyperULat = hgnn(uuHyper, lats[-1][:N_USERS], V)
        hyperILat = hgnn(iiHyper, lats[-1][N_USERS:], V)
        gnnLats.append(temEmbeds)
        hyperLats.append(jnp.concatenate([hyperULat, hyperILat], axis=0))
        lats.append(_padd(temEmbeds, hyperLats[-1]))
    out = lats[0]
    for l in lats[1:]:
        out = _padd(out, l)
    return (out, tuple(gnnLats), tuple(hyperLats))


# speedup vs baseline: 6.9530x; 6.9530x over previous
import jax
import jax.numpy as jnp
from jax import lax
from jax.experimental import pallas as pl
from jax.experimental.pallas import tpu as pltpu
from jax.experimental.pallas import tpu_sc as plsc

N_USERS = 25000
N_ITEMS = 25000
N = N_USERS + N_ITEMS
EMB = 64
HYP = 128
C_LAYERS = 2
SLOPE = 0.1
E = 800000

F2 = EMB // 2
NC = 2
NS = 16
SUB = 128
NSUB = 5
CH = SUB * NSUB
NCHUNKS = E // CH
CPS = -(-NCHUNKS // NS)
NPAD = 50048
ROWS_PER_SUB = NPAD // NS

BLK = 1000
NBLK = N // BLK
NBLK_H = N_USERS // BLK


def _leaky(x):
    return jnp.where(x >= 0, x, SLOPE * x)


def _spmm_body(x2, rows, cols, vals, zeros, out,
               acc, col_v, row_v, val_v, msg, lsem, gsem, ssem):
    c = lax.axis_index("c")
    s = lax.axis_index("s")

    pltpu.sync_copy(zeros.at[pl.ds(s * ROWS_PER_SUB, ROWS_PER_SUB)],
                    acc.at[pl.ds(s * ROWS_PER_SUB, ROWS_PER_SUB)])
    plsc.subcore_barrier()

    coff = jnp.full((16,), c * N, dtype=jnp.int32)

    def chunk(gi, _):
        g = gi * NS + s

        @pl.when(g < NCHUNKS)
        def _():
            base = g * CH
            lds = [pltpu.async_copy(rows.at[pl.ds(base + j * SUB, SUB)],
                                    row_v.at[j], lsem)
                   for j in range(NSUB)]
            d1 = pltpu.async_copy(cols.at[pl.ds(base, CH)], col_v, lsem)
            d2 = pltpu.async_copy(vals.at[pl.ds(base, CH)], val_v, lsem)
            for d in lds:
                d.wait()
            d1.wait(); d2.wait()
            for k in range(CH // 16):
                col_v[pl.ds(k * 16, 16)] = col_v[pl.ds(k * 16, 16)] + coff
            gds = [pltpu.async_copy(x2.at[col_v.at[pl.ds(j * SUB, SUB)]],
                                    msg.at[pl.ds(j * SUB, SUB)], gsem)
                   for j in range(NSUB)]
            for d in gds:
                d.wait()

            def scale(G, carry):
                vv = val_v[pl.ds(G * 16, 16)]
                for t in range(16):
                    bv = lax.broadcast(vv[t], (16,))
                    e = G * 16 + t
                    msg[e, pl.ds(0, 16)] = msg[e, pl.ds(0, 16)] * bv
                    msg[e, pl.ds(16, 16)] = msg[e, pl.ds(16, 16)] * bv
                return carry
            lax.fori_loop(0, CH // 16, scale, None)

            sds = [pltpu.async_copy(msg.at[pl.ds(j * SUB, SUB)],
                                    acc.at[row_v.at[j]], ssem, add=True)
                   for j in range(NSUB)]
            for d in sds:
                d.wait()
        return _

    lax.fori_loop(0, CPS, chunk, None)
    plsc.subcore_barrier()

    pltpu.sync_copy(acc.at[pl.ds(s * ROWS_PER_SUB, ROWS_PER_SUB)],
                    out.at[pl.ds(c * NPAD + s * ROWS_PER_SUB, ROWS_PER_SUB)])


_spmm = pl.kernel(
    _spmm_body,
    out_type=jax.ShapeDtypeStruct((2 * NPAD, F2), jnp.float32),
    mesh=plsc.VectorSubcoreMesh(core_axis_name="c", subcore_axis_name="s",
                                num_cores=NC, num_subcores=NS),
    compiler_params=pltpu.CompilerParams(use_tc_tiling_on_sc=False),
    scratch_types=[
        pltpu.VMEM_SHARED((NPAD, F2), jnp.float32),
        pltpu.VMEM((CH,), jnp.int32),
        pltpu.VMEM((NSUB, SUB), jnp.int32),
        pltpu.VMEM((CH,), jnp.float32),
        pltpu.VMEM((CH, F2), jnp.float32),
        pltpu.SemaphoreType.DMA,
        pltpu.SemaphoreType.DMA,
        pltpu.SemaphoreType.DMA,
    ],
)


def _hypercat_body(x_ref, uH_ref, iH_ref, o_ref):
    i = pl.program_id(0)
    W = jnp.where(i < NBLK_H, uH_ref[...], iH_ref[...])
    o_ref[...] = jnp.dot(x_ref[...], W, preferred_element_type=jnp.float32)


def _hypercat(x0, uHyper, iHyper):
    return pl.pallas_call(
        _hypercat_body,
        grid=(NBLK,),
        in_specs=[
            pl.BlockSpec((BLK, EMB), lambda i: (i, 0)),
            pl.BlockSpec((EMB, HYP), lambda i: (0, 0)),
            pl.BlockSpec((EMB, HYP), lambda i: (0, 0)),
        ],
        out_specs=pl.BlockSpec((BLK, HYP), lambda i: (i, 0)),
        out_shape=jax.ShapeDtypeStruct((N, HYP), jnp.float32),
    )(x0, uHyper, iHyper)


def _latf_body(hc_ref, x_ref, V_ref, o_ref, acc_ref):
    j = pl.program_id(1)
    part = lax.dot_general(hc_ref[...], x_ref[...],
                           (((0,), (0,)), ((), ())),
                           preferred_element_type=jnp.float32)

    @pl.when(j == 0)
    def _():
        acc_ref[...] = part

    @pl.when(j > 0)
    def _():
        acc_ref[...] = acc_ref[...] + part

    @pl.when(j == NBLK_H - 1)
    def _():
        lat = _leaky(acc_ref[...])
        for _ in range(C_LAYERS):
            lat = _leaky(jnp.dot(V_ref[...], lat,
                                 preferred_element_type=jnp.float32)) + lat
        o_ref[0] = lat


def _latf(hyperCat, x, V):
    return pl.pallas_call(
        _latf_body,
        grid=(2, NBLK_H),
        in_specs=[
            pl.BlockSpec((BLK, HYP), lambda h, j: (h * NBLK_H + j, 0)),
            pl.BlockSpec((BLK, EMB), lambda h, j: (h * NBLK_H + j, 0)),
            pl.BlockSpec((HYP, HYP), lambda h, j: (0, 0)),
        ],
        out_specs=pl.BlockSpec((1, HYP, EMB), lambda h, j: (h, 0, 0)),
        out_shape=jax.ShapeDtypeStruct((2, HYP, EMB), jnp.float32),
        scratch_shapes=[pltpu.VMEM((HYP, EMB), jnp.float32)],
    )(hyperCat, x, V)


def _combine0_body(s2_ref, hc_ref, latf_ref, gnn_ref, hyp_ref, xn_ref, xns_ref):
    sb = jnp.concatenate([s2_ref[0], s2_ref[1]], axis=1)
    g = _leaky(sb)
    hypb = _leaky(jnp.dot(hc_ref[...], latf_ref[0],
                          preferred_element_type=jnp.float32))
    gnn_ref[...] = g
    hyp_ref[...] = hypb
    xn = g + hypb
    xn_ref[...] = xn
    xns_ref[0] = xn[:, :F2]
    xns_ref[1] = xn[:, F2:]


def _combine0(s2, hyperCat, latf):
    return pl.pallas_call(
        _combine0_body,
        grid=(NBLK,),
        in_specs=[
            pl.BlockSpec((2, BLK, F2), lambda i: (0, i, 0)),
            pl.BlockSpec((BLK, HYP), lambda i: (i, 0)),
            pl.BlockSpec((1, HYP, EMB), lambda i: (i // NBLK_H, 0, 0)),
        ],
        out_specs=[
            pl.BlockSpec((BLK, EMB), lambda i: (i, 0)),
            pl.BlockSpec((BLK, EMB), lambda i: (i, 0)),
            pl.BlockSpec((BLK, EMB), lambda i: (i, 0)),
            pl.BlockSpec((2, BLK, F2), lambda i: (0, i, 0)),
        ],
        out_shape=[
            jax.ShapeDtypeStruct((N, EMB), jnp.float32),
            jax.ShapeDtypeStruct((N, EMB), jnp.float32),
            jax.ShapeDtypeStruct((N, EMB), jnp.float32),
            jax.ShapeDtypeStruct((2, N, F2), jnp.float32),
        ],
    )(s2, hyperCat, latf)


def _combine1_body(s2_ref, hc_ref, latf_ref, x0_ref, x1_ref,
                   gnn_ref, hyp_ref, out_ref):
    sb = jnp.concatenate([s2_ref[0], s2_ref[1]], axis=1)
    g = _leaky(sb)
    hypb = _leaky(jnp.dot(hc_ref[...], latf_ref[0],
                          preferred_element_type=jnp.float32))
    gnn_ref[...] = g
    hyp_ref[...] = hypb
    out_ref[...] = x0_ref[...] + x1_ref[...] + g + hypb


def _combine1(s2, hyperCat, latf, x0, x1):
    return pl.pallas_call(
        _combine1_body,
        grid=(NBLK,),
        in_specs=[
            pl.BlockSpec((2, BLK, F2), lambda i: (0, i, 0)),
            pl.BlockSpec((BLK, HYP), lambda i: (i, 0)),
            pl.BlockSpec((1, HYP, EMB), lambda i: (i // NBLK_H, 0, 0)),
            pl.BlockSpec((BLK, EMB), lambda i: (i, 0)),
            pl.BlockSpec((BLK, EMB), lambda i: (i, 0)),
        ],
        out_specs=[
            pl.BlockSpec((BLK, EMB), lambda i: (i, 0)),
            pl.BlockSpec((BLK, EMB), lambda i: (i, 0)),
            pl.BlockSpec((BLK, EMB), lambda i: (i, 0)),
        ],
        out_shape=[
            jax.ShapeDtypeStruct((N, EMB), jnp.float32),
            jax.ShapeDtypeStruct((N, EMB), jnp.float32),
            jax.ShapeDtypeStruct((N, EMB), jnp.float32),
        ],
    )(s2, hyperCat, latf, x0, x1)


def kernel(adj_indices, adj_values, uEmbeds, iEmbeds, uHyper, iHyper, V, keepRate):
    x0 = jnp.concatenate([uEmbeds, iEmbeds], axis=0)
    x0s = jnp.concatenate([x0[:, :F2], x0[:, F2:]], axis=0)
    rows = adj_indices[0]
    cols = adj_indices[1]
    zeros = jnp.zeros((NPAD, F2), jnp.float32)

    def stacked_view(s):
        sp = s.reshape(2, NPAD, F2)
        return jnp.stack([sp[0, :N], sp[1, :N]])

    hyperCat = _hypercat(x0, uHyper, iHyper)

    s0 = _spmm(x0s, rows, cols, adj_values, zeros)
    latf0 = _latf(hyperCat, x0, V)
    gnn0, hyp0, x1, x1s = _combine0(stacked_view(s0), hyperCat, latf0)

    s1 = _spmm(x1s.reshape(2 * N, F2), rows, cols, adj_values, zeros)
    latf1 = _latf(hyperCat, x1, V)
    gnn1, hyp1, out = _combine1(stacked_view(s1), hyperCat, latf1, x0, x1)

    return (out, (gnn0, gnn1), (hyp0, hyp1))


# speedup vs baseline: 8.9538x; 1.2878x over previous
import jax
import jax.numpy as jnp
from jax import lax
from jax.experimental import pallas as pl
from jax.experimental.pallas import tpu as pltpu
from jax.experimental.pallas import tpu_sc as plsc

N_USERS = 25000
N_ITEMS = 25000
N = N_USERS + N_ITEMS
EMB = 64
HYP = 128
C_LAYERS = 2
SLOPE = 0.1
E = 800000

F2 = EMB // 2
NC = 2
NS = 16
SUB = 128
NSUB = 3
CH = SUB * NSUB
CPS = 132
E_PAD = NS * CPS * CH
NPAD = 50048
ROWS_PER_SUB = NPAD // NS

BLK = 1000
NBLK = N // BLK
NBLK_H = N_USERS // BLK


def _leaky(x):
    return jnp.where(x >= 0, x, SLOPE * x)


def _spmm_body(x2, rows, cols, vals, zeros, out, acc,
               col0, col1, val0, val1, row0, row1, row2, row3, msg0, msg1,
               lsem0, lsem1, gsem0, gsem1, ssem0, ssem1):
    c_ax = lax.axis_index("c")
    s_ax = lax.axis_index("s")

    cols_b = [col0, col1]
    vals_b = [val0, val1]
    rows_b = [row0, row1, row2, row3]
    msgs_b = [msg0, msg1]
    lsems = [lsem0, lsem1]
    gsems = [gsem0, gsem1]
    ssems = [ssem0, ssem1]

    pltpu.sync_copy(zeros.at[pl.ds(s_ax * ROWS_PER_SUB, ROWS_PER_SUB)],
                    acc.at[pl.ds(s_ax * ROWS_PER_SUB, ROWS_PER_SUB)])
    plsc.subcore_barrier()

    coff = jnp.full((16,), c_ax * N, dtype=jnp.int32)

    def l_descs(c, b, r, sem_ok=True):
        base = (s_ax * CPS + c) * CH
        ds = [pltpu.make_async_copy(cols.at[pl.ds(base, CH)], cols_b[b], lsems[b]),
              pltpu.make_async_copy(vals.at[pl.ds(base, CH)], vals_b[b], lsems[b])]
        for j in range(NSUB):
            ds.append(pltpu.make_async_copy(
                rows.at[pl.ds(base + j * SUB, SUB)], rows_b[r].at[j], lsems[b]))
        return ds

    def g_descs(c, b):
        return [pltpu.make_async_copy(
                    x2.at[cols_b[b].at[pl.ds(j * SUB, SUB)]],
                    msgs_b[b].at[pl.ds(j * SUB, SUB)], gsems[b])
                for j in range(NSUB)]

    def w_start(c, b, r):
        for j in range(NSUB):
            pltpu.async_copy(msgs_b[b].at[pl.ds(j * SUB, SUB)],
                             acc.at[rows_b[r].at[j]], ssems[b], add=True)

    def w_wait(c, b, r):
        for j in range(NSUB):
            pltpu.make_async_copy(msgs_b[b].at[pl.ds(j * SUB, SUB)],
                                  acc.at[rows_b[r].at[j]], ssems[b]).wait()

    def adjust(b):
        def body(k, carry):
            cb = cols_b[b]
            cb[pl.ds(k * 16, 16)] = cb[pl.ds(k * 16, 16)] + coff
            return carry
        lax.fori_loop(0, CH // 16, body, None, unroll=4)

    def scale(b):
        mb = msgs_b[b]
        vb = vals_b[b]

        def body(G, carry):
            vv = vb[pl.ds(G * 16, 16)]
            for t in range(16):
                bv = lax.broadcast(vv[t], (16,))
                e = G * 16 + t
                mb[e, pl.ds(0, 16)] = mb[e, pl.ds(0, 16)] * bv
                mb[e, pl.ds(16, 16)] = mb[e, pl.ds(16, 16)] * bv
            return carry
        lax.fori_loop(0, CH // 16, body, None)

    def chunk_step(c, q):
        b = q % 2
        r = q % 4

        @pl.when(c >= 2)
        def _():
            w_wait(c - 2, b, r)
        for d in g_descs(c, b):
            d.start()

        @pl.when(c >= 1)
        def _():
            for d in g_descs(c - 1, 1 - b):
                d.wait()
            scale(1 - b)
            w_start(c - 1, 1 - b, (q + 3) % 4)

        @pl.when(c + 1 < CPS)
        def _():
            nds = l_descs(c + 1, 1 - b, (q + 1) % 4)
            for d in nds:
                d.start()
            for d in nds:
                d.wait()
            adjust(1 - b)

    p = l_descs(0, 0, 0)
    for d in p:
        d.start()
    for d in p:
        d.wait()
    adjust(0)

    def quad(t, carry):
        for q in range(4):
            chunk_step(t * 4 + q, q)
        return carry
    lax.fori_loop(0, CPS // 4, quad, None)

    last = CPS - 1
    for d in g_descs(last, last % 2):
        d.wait()
    scale(last % 2)
    w_start(last, last % 2, last % 4)
    w_wait(last - 1, (last - 1) % 2, (last - 1) % 4)
    w_wait(last, last % 2, last % 4)

    plsc.subcore_barrier()

    pltpu.sync_copy(acc.at[pl.ds(s_ax * ROWS_PER_SUB, ROWS_PER_SUB)],
                    out.at[pl.ds(c_ax * NPAD + s_ax * ROWS_PER_SUB, ROWS_PER_SUB)])


_spmm = pl.kernel(
    _spmm_body,
    out_type=jax.ShapeDtypeStruct((2 * NPAD, F2), jnp.float32),
    mesh=plsc.VectorSubcoreMesh(core_axis_name="c", subcore_axis_name="s",
                                num_cores=NC, num_subcores=NS),
    compiler_params=pltpu.CompilerParams(use_tc_tiling_on_sc=False),
    scratch_types=[
        pltpu.VMEM_SHARED((NPAD, F2), jnp.float32),
        pltpu.VMEM((CH,), jnp.int32),
        pltpu.VMEM((CH,), jnp.int32),
        pltpu.VMEM((CH,), jnp.float32),
        pltpu.VMEM((CH,), jnp.float32),
        pltpu.VMEM((NSUB, SUB), jnp.int32),
        pltpu.VMEM((NSUB, SUB), jnp.int32),
        pltpu.VMEM((NSUB, SUB), jnp.int32),
        pltpu.VMEM((NSUB, SUB), jnp.int32),
        pltpu.VMEM((CH, F2), jnp.float32),
        pltpu.VMEM((CH, F2), jnp.float32),
        pltpu.SemaphoreType.DMA,
        pltpu.SemaphoreType.DMA,
        pltpu.SemaphoreType.DMA,
        pltpu.SemaphoreType.DMA,
        pltpu.SemaphoreType.DMA,
        pltpu.SemaphoreType.DMA,
    ],
)


def _gram_body(x0_ref, xk_ref, uH_ref, iH_ref, V_ref, m_ref, acc_ref):
    h = pl.program_id(0)
    j = pl.program_id(1)
    part = lax.dot_general(x0_ref[...], xk_ref[...],
                           (((0,), (0,)), ((), ())),
                           preferred_element_type=jnp.float32)

    @pl.when(j == 0)
    def _():
        acc_ref[...] = part

    @pl.when(j > 0)
    def _():
        acc_ref[...] = acc_ref[...] + part

    @pl.when(j == NBLK_H - 1)
    def _():
        W = jnp.where(h == 0, uH_ref[...], iH_ref[...])
        lat = _leaky(lax.dot_general(W, acc_ref[...],
                                     (((0,), (0,)), ((), ())),
                                     preferred_element_type=jnp.float32))
        for _ in range(C_LAYERS):
            lat = _leaky(jnp.dot(V_ref[...], lat,
                                 preferred_element_type=jnp.float32)) + lat
        m_ref[0] = jnp.dot(W, lat, preferred_element_type=jnp.float32)


def _gram(x0, xk, uHyper, iHyper, V):
    return pl.pallas_call(
        _gram_body,
        grid=(2, NBLK_H),
        in_specs=[
            pl.BlockSpec((BLK, EMB), lambda h, j: (h * NBLK_H + j, 0)),
            pl.BlockSpec((BLK, EMB), lambda h, j: (h * NBLK_H + j, 0)),
            pl.BlockSpec((EMB, HYP), lambda h, j: (0, 0)),
            pl.BlockSpec((EMB, HYP), lambda h, j: (0, 0)),
            pl.BlockSpec((HYP, HYP), lambda h, j: (0, 0)),
        ],
        out_specs=pl.BlockSpec((1, EMB, EMB), lambda h, j: (h, 0, 0)),
        out_shape=jax.ShapeDtypeStruct((2, EMB, EMB), jnp.float32),
        scratch_shapes=[pltpu.VMEM((EMB, EMB), jnp.float32)],
    )(x0, xk, uHyper, iHyper, V)


def _combine0_body(s2_ref, x0_ref, m_ref, gnn_ref, hyp_ref, xn_ref, xns_ref):
    sb = jnp.concatenate([s2_ref[0], s2_ref[1]], axis=1)
    g = _leaky(sb)
    hypb = _leaky(jnp.dot(x0_ref[...], m_ref[0],
                          preferred_element_type=jnp.float32))
    gnn_ref[...] = g
    hyp_ref[...] = hypb
    xn = g + hypb
    xn_ref[...] = xn
    xns_ref[0] = xn[:, :F2]
    xns_ref[1] = xn[:, F2:]


def _combine0(s2, x0, m):
    return pl.pallas_call(
        _combine0_body,
        grid=(NBLK,),
        in_specs=[
            pl.BlockSpec((2, BLK, F2), lambda i: (0, i, 0)),
            pl.BlockSpec((BLK, EMB), lambda i: (i, 0)),
            pl.BlockSpec((1, EMB, EMB), lambda i: (i // NBLK_H, 0, 0)),
        ],
        out_specs=[
            pl.BlockSpec((BLK, EMB), lambda i: (i, 0)),
            pl.BlockSpec((BLK, EMB), lambda i: (i, 0)),
            pl.BlockSpec((BLK, EMB), lambda i: (i, 0)),
            pl.BlockSpec((2, BLK, F2), lambda i: (0, i, 0)),
        ],
        out_shape=[
            jax.ShapeDtypeStruct((N, EMB), jnp.float32),
            jax.ShapeDtypeStruct((N, EMB), jnp.float32),
            jax.ShapeDtypeStruct((N, EMB), jnp.float32),
            jax.ShapeDtypeStruct((2, N, F2), jnp.float32),
        ],
    )(s2, x0, m)


def _combine1_body(s2_ref, x0_ref, x1_ref, m_ref, gnn_ref, hyp_ref, out_ref):
    sb = jnp.concatenate([s2_ref[0], s2_ref[1]], axis=1)
    g = _leaky(sb)
    hypb = _leaky(jnp.dot(x0_ref[...], m_ref[0],
                          preferred_element_type=jnp.float32))
    gnn_ref[...] = g
    hyp_ref[...] = hypb
    out_ref[...] = x0_ref[...] + x1_ref[...] + g + hypb


def _combine1(s2, x0, x1, m):
    return pl.pallas_call(
        _combine1_body,
        grid=(NBLK,),
        in_specs=[
            pl.BlockSpec((2, BLK, F2), lambda i: (0, i, 0)),
            pl.BlockSpec((BLK, EMB), lambda i: (i, 0)),
            pl.BlockSpec((BLK, EMB), lambda i: (i, 0)),
            pl.BlockSpec((1, EMB, EMB), lambda i: (i // NBLK_H, 0, 0)),
        ],
        out_specs=[
            pl.BlockSpec((BLK, EMB), lambda i: (i, 0)),
            pl.BlockSpec((BLK, EMB), lambda i: (i, 0)),
            pl.BlockSpec((BLK, EMB), lambda i: (i, 0)),
        ],
        out_shape=[
            jax.ShapeDtypeStruct((N, EMB), jnp.float32),
            jax.ShapeDtypeStruct((N, EMB), jnp.float32),
            jax.ShapeDtypeStruct((N, EMB), jnp.float32),
        ],
    )(s2, x0, x1, m)


def kernel(adj_indices, adj_values, uEmbeds, iEmbeds, uHyper, iHyper, V, keepRate):
    x0 = jnp.concatenate([uEmbeds, iEmbeds], axis=0)
    x0s = jnp.concatenate([x0[:, :F2], x0[:, F2:]], axis=0)

    npad_e = E_PAD - E
    ar = jnp.arange(npad_e, dtype=jnp.int32)
    rows = jnp.concatenate([adj_indices[0], N + ar % (NPAD - N)])
    cols = jnp.concatenate([adj_indices[1], ar % N])
    vals = jnp.concatenate([adj_values, jnp.zeros((npad_e,), jnp.float32)])
    zeros = jnp.zeros((NPAD, F2), jnp.float32)

    def stacked_view(s):
        sp = s.reshape(2, NPAD, F2)
        return jnp.stack([sp[0, :N], sp[1, :N]])

    s0 = _spmm(x0s, rows, cols, vals, zeros)
    m0 = _gram(x0, x0, uHyper, iHyper, V)
    gnn0, hyp0, x1, x1s = _combine0(stacked_view(s0), x0, m0)

    s1 = _spmm(x1s.reshape(2 * N, F2), rows, cols, vals, zeros)
    m1 = _gram(x0, x1, uHyper, iHyper, V)
    gnn1, hyp1, out = _combine1(stacked_view(s1), x0, x1, m1)

    return (out, (gnn0, gnn1), (hyp0, hyp1))


# speedup vs baseline: 10.1904x; 1.1381x over previous
import jax
import jax.numpy as jnp
import numpy as np
from jax import lax
from jax.experimental import pallas as pl
from jax.experimental.pallas import tpu as pltpu
from jax.experimental.pallas import tpu_sc as plsc

N_USERS = 25000
N_ITEMS = 25000
N = N_USERS + N_ITEMS
EMB = 64
HYP = 128
C_LAYERS = 2
SLOPE = 0.1
E = 800000

F2 = EMB // 2
NC = 2
NS = 16
SUB = 128
NSUB = 3
CH = SUB * NSUB
CPS = 132
E_PAD = NS * CPS * CH
NPAD = 50048
ROWS_PER_SUB = NPAD // NS

BLK = 5000
NBLK = N // BLK
NBLK_H = N_USERS // BLK


def _leaky(x):
    return jnp.where(x >= 0, x, SLOPE * x)


def _spmm_body(x2, rows, cols, vals, zeros, out, acc,
               col0, col1, val0, val1, row0, row1, row2, row3, msg0, msg1,
               lsem0, lsem1, gsem0, gsem1, ssem0, ssem1):
    c_ax = lax.axis_index("c")
    s_ax = lax.axis_index("s")
    xh = x2.at[c_ax]

    cols_b = [col0, col1]
    vals_b = [val0, val1]
    rows_b = [row0, row1, row2, row3]
    msgs_b = [msg0, msg1]
    lsems = [lsem0, lsem1]
    gsems = [gsem0, gsem1]
    ssems = [ssem0, ssem1]

    pltpu.sync_copy(zeros.at[pl.ds(s_ax * ROWS_PER_SUB, ROWS_PER_SUB)],
                    acc.at[pl.ds(s_ax * ROWS_PER_SUB, ROWS_PER_SUB)])
    plsc.subcore_barrier()

    def l_descs(c, b, r):
        base = (s_ax * CPS + c) * CH
        ds = [pltpu.make_async_copy(cols.at[pl.ds(base, CH)], cols_b[b], lsems[b]),
              pltpu.make_async_copy(vals.at[pl.ds(base, CH)], vals_b[b], lsems[b])]
        for j in range(NSUB):
            ds.append(pltpu.make_async_copy(
                rows.at[pl.ds(base + j * SUB, SUB)], rows_b[r].at[j], lsems[b]))
        return ds

    def g_descs(c, b):
        return [pltpu.make_async_copy(
                    xh.at[cols_b[b].at[pl.ds(j * SUB, SUB)]],
                    msgs_b[b].at[pl.ds(j * SUB, SUB)], gsems[b])
                for j in range(NSUB)]

    def w_start(c, b, r):
        for j in range(NSUB):
            pltpu.async_copy(msgs_b[b].at[pl.ds(j * SUB, SUB)],
                             acc.at[rows_b[r].at[j]], ssems[b], add=True)

    def w_wait(c, b, r):
        for j in range(NSUB):
            pltpu.make_async_copy(msgs_b[b].at[pl.ds(j * SUB, SUB)],
                                  acc.at[rows_b[r].at[j]], ssems[b]).wait()

    def scale(b):
        mb = msgs_b[b]
        vb = vals_b[b]

        def body(G, carry):
            vv = vb[pl.ds(G * 16, 16)]
            for t in range(16):
                bv = lax.broadcast(vv[t], (16,))
                e = G * 16 + t
                mb[e, pl.ds(0, 16)] = mb[e, pl.ds(0, 16)] * bv
                mb[e, pl.ds(16, 16)] = mb[e, pl.ds(16, 16)] * bv
            return carry
        lax.fori_loop(0, CH // 16, body, None)

    def chunk_step(c, q):
        b = q % 2
        r = q % 4

        @pl.when(c >= 2)
        def _():
            w_wait(c - 2, b, r)
        for d in g_descs(c, b):
            d.start()

        @pl.when(c >= 1)
        def _():
            for d in g_descs(c - 1, 1 - b):
                d.wait()
            scale(1 - b)
            w_start(c - 1, 1 - b, (q + 3) % 4)

        @pl.when(c + 1 < CPS)
        def _():
            nds = l_descs(c + 1, 1 - b, (q + 1) % 4)
            for d in nds:
                d.start()
            for d in nds:
                d.wait()

    p = l_descs(0, 0, 0)
    for d in p:
        d.start()
    for d in p:
        d.wait()

    def quad(t, carry):
        for q in range(4):
            chunk_step(t * 4 + q, q)
        return carry
    lax.fori_loop(0, CPS // 4, quad, None)

    last = CPS - 1
    for d in g_descs(last, last % 2):
        d.wait()
    scale(last % 2)
    w_start(last, last % 2, last % 4)
    w_wait(last - 1, (last - 1) % 2, (last - 1) % 4)
    w_wait(last, last % 2, last % 4)

    plsc.subcore_barrier()

    pltpu.sync_copy(acc.at[pl.ds(s_ax * ROWS_PER_SUB, ROWS_PER_SUB)],
                    out.at[pl.ds(s_ax * ROWS_PER_SUB, ROWS_PER_SUB),
                           pl.ds(c_ax * F2, F2)])


_spmm = pl.kernel(
    _spmm_body,
    out_type=jax.ShapeDtypeStruct((NPAD, EMB), jnp.float32),
    mesh=plsc.VectorSubcoreMesh(core_axis_name="c", subcore_axis_name="s",
                                num_cores=NC, num_subcores=NS),
    compiler_params=pltpu.CompilerParams(use_tc_tiling_on_sc=False),
    scratch_types=[
        pltpu.VMEM_SHARED((NPAD, F2), jnp.float32),
        pltpu.VMEM((CH,), jnp.int32),
        pltpu.VMEM((CH,), jnp.int32),
        pltpu.VMEM((CH,), jnp.float32),
        pltpu.VMEM((CH,), jnp.float32),
        pltpu.VMEM((NSUB, SUB), jnp.int32),
        pltpu.VMEM((NSUB, SUB), jnp.int32),
        pltpu.VMEM((NSUB, SUB), jnp.int32),
        pltpu.VMEM((NSUB, SUB), jnp.int32),
        pltpu.VMEM((CH, F2), jnp.float32),
        pltpu.VMEM((CH, F2), jnp.float32),
        pltpu.SemaphoreType.DMA,
        pltpu.SemaphoreType.DMA,
        pltpu.SemaphoreType.DMA,
        pltpu.SemaphoreType.DMA,
        pltpu.SemaphoreType.DMA,
        pltpu.SemaphoreType.DMA,
    ],
)


def _gram_body(x0_ref, xk_ref, uH_ref, iH_ref, V_ref, m_ref, acc_ref):
    h = pl.program_id(0)
    j = pl.program_id(1)
    part = lax.dot_general(x0_ref[...], xk_ref[...],
                           (((0,), (0,)), ((), ())),
                           preferred_element_type=jnp.float32)

    @pl.when(j == 0)
    def _():
        acc_ref[...] = part

    @pl.when(j > 0)
    def _():
        acc_ref[...] = acc_ref[...] + part

    @pl.when(j == NBLK_H - 1)
    def _():
        W = jnp.where(h == 0, uH_ref[...], iH_ref[...])
        lat = _leaky(lax.dot_general(W, acc_ref[...],
                                     (((0,), (0,)), ((), ())),
                                     preferred_element_type=jnp.float32))
        for _ in range(C_LAYERS):
            lat = _leaky(jnp.dot(V_ref[...], lat,
                                 preferred_element_type=jnp.float32)) + lat
        m_ref[0] = jnp.dot(W, lat, preferred_element_type=jnp.float32)


def _gram(x0, xk, uHyper, iHyper, V):
    return pl.pallas_call(
        _gram_body,
        grid=(2, NBLK_H),
        in_specs=[
            pl.BlockSpec((BLK, EMB), lambda h, j: (h * NBLK_H + j, 0)),
            pl.BlockSpec((BLK, EMB), lambda h, j: (h * NBLK_H + j, 0)),
            pl.BlockSpec((EMB, HYP), lambda h, j: (0, 0)),
            pl.BlockSpec((EMB, HYP), lambda h, j: (0, 0)),
            pl.BlockSpec((HYP, HYP), lambda h, j: (0, 0)),
        ],
        out_specs=pl.BlockSpec((1, EMB, EMB), lambda h, j: (h, 0, 0)),
        out_shape=jax.ShapeDtypeStruct((2, EMB, EMB), jnp.float32),
        scratch_shapes=[pltpu.VMEM((EMB, EMB), jnp.float32)],
    )(x0, xk, uHyper, iHyper, V)


def _combine0_body(s_ref, x0_ref, m_ref, gnn_ref, hyp_ref, xn_ref, xp_ref):
    g = _leaky(s_ref[...])
    hypb = _leaky(jnp.dot(x0_ref[...], m_ref[0],
                          preferred_element_type=jnp.float32))
    gnn_ref[...] = g
    hyp_ref[...] = hypb
    xn = g + hypb
    xn_ref[...] = xn
    xp_ref[0] = xn[:, :F2]
    xp_ref[1] = xn[:, F2:]


def _combine0(s, x0, m):
    return pl.pallas_call(
        _combine0_body,
        grid=(NBLK,),
        in_specs=[
            pl.BlockSpec((BLK, EMB), lambda i: (i, 0)),
            pl.BlockSpec((BLK, EMB), lambda i: (i, 0)),
            pl.BlockSpec((1, EMB, EMB), lambda i: (i // NBLK_H, 0, 0)),
        ],
        out_specs=[
            pl.BlockSpec((BLK, EMB), lambda i: (i, 0)),
            pl.BlockSpec((BLK, EMB), lambda i: (i, 0)),
            pl.BlockSpec((BLK, EMB), lambda i: (i, 0)),
            pl.BlockSpec((2, BLK, F2), lambda i: (0, i, 0)),
        ],
        out_shape=[
            jax.ShapeDtypeStruct((N, EMB), jnp.float32),
            jax.ShapeDtypeStruct((N, EMB), jnp.float32),
            jax.ShapeDtypeStruct((N, EMB), jnp.float32),
            jax.ShapeDtypeStruct((2, N, F2), jnp.float32),
        ],
    )(s, x0, m)


def _combine1_body(s_ref, x0_ref, x1_ref, m_ref, gnn_ref, hyp_ref, out_ref):
    g = _leaky(s_ref[...])
    hypb = _leaky(jnp.dot(x0_ref[...], m_ref[0],
                          preferred_element_type=jnp.float32))
    gnn_ref[...] = g
    hyp_ref[...] = hypb
    out_ref[...] = x0_ref[...] + x1_ref[...] + g + hypb


def _combine1(s, x0, x1, m):
    return pl.pallas_call(
        _combine1_body,
        grid=(NBLK,),
        in_specs=[
            pl.BlockSpec((BLK, EMB), lambda i: (i, 0)),
            pl.BlockSpec((BLK, EMB), lambda i: (i, 0)),
            pl.BlockSpec((BLK, EMB), lambda i: (i, 0)),
            pl.BlockSpec((1, EMB, EMB), lambda i: (i // NBLK_H, 0, 0)),
        ],
        out_specs=[
            pl.BlockSpec((BLK, EMB), lambda i: (i, 0)),
            pl.BlockSpec((BLK, EMB), lambda i: (i, 0)),
            pl.BlockSpec((BLK, EMB), lambda i: (i, 0)),
        ],
        out_shape=[
            jax.ShapeDtypeStruct((N, EMB), jnp.float32),
            jax.ShapeDtypeStruct((N, EMB), jnp.float32),
            jax.ShapeDtypeStruct((N, EMB), jnp.float32),
        ],
    )(s, x0, x1, m)


_npe = E_PAD - E
_ar = np.arange(_npe)
_ROWS_TAIL = np.asarray(N + _ar % (NPAD - N), dtype=np.int32)
_COLS_TAIL = np.asarray(_ar % N, dtype=np.int32)
_VALS_TAIL = np.zeros((_npe,), dtype=np.float32)


def kernel(adj_indices, adj_values, uEmbeds, iEmbeds, uHyper, iHyper, V, keepRate):
    x0 = jnp.concatenate([uEmbeds, iEmbeds], axis=0)
    x0s = jnp.stack([x0[:, :F2], x0[:, F2:]])

    rows = jnp.concatenate([adj_indices[0], jnp.asarray(_ROWS_TAIL)])
    cols = jnp.concatenate([adj_indices[1], jnp.asarray(_COLS_TAIL)])
    vals = jnp.concatenate([adj_values, jnp.asarray(_VALS_TAIL)])
    zeros = jnp.zeros((NPAD, F2), jnp.float32)

    s0 = _spmm(x0s, rows, cols, vals, zeros)
    m0 = _gram(x0, x0, uHyper, iHyper, V)
    gnn0, hyp0, x1, x1p = _combine0(s0, x0, m0)

    s1 = _spmm(x1p, rows, cols, vals, zeros)
    m1 = _gram(x0, x1, uHyper, iHyper, V)
    gnn1, hyp1, out = _combine1(s1, x0, x1, m1)

    return (out, (gnn0, gnn1), (hyp0, hyp1))


# speedup vs baseline: 11.4602x; 1.1246x over previous
import jax
import jax.numpy as jnp
import numpy as np
from jax import lax
from jax.experimental import pallas as pl
from jax.experimental.pallas import tpu as pltpu
from jax.experimental.pallas import tpu_sc as plsc

N_USERS = 25000
N_ITEMS = 25000
N = N_USERS + N_ITEMS
EMB = 64
HYP = 128
C_LAYERS = 2
SLOPE = 0.1
E = 800000

F2 = EMB // 2
NC = 2
NS = 16
SUB = 128
NSUB = 3
CH = SUB * NSUB
CPS = 132
E_PAD = NS * CPS * CH
NPAD = 50048
ROWS_PER_SUB = NPAD // NS

BLK = 5000
NBLK = N // BLK
NBLK_H = N_USERS // BLK


def _leaky(x):
    return jnp.where(x >= 0, x, SLOPE * x)


def _spmm_body(x2, rows, cols, vals, zeros, out, acc,
               col0, col1, val0, val1, row0, row1, row2, row3, msg0, msg1,
               lsem0, lsem1, gsem0, gsem1, ssem0, ssem1):
    c_ax = lax.axis_index("c")
    s_ax = lax.axis_index("s")

    cols_b = [col0, col1]
    vals_b = [val0, val1]
    rows_b = [row0, row1, row2, row3]
    msgs_b = [msg0, msg1]
    lsems = [lsem0, lsem1]
    gsems = [gsem0, gsem1]
    ssems = [ssem0, ssem1]

    pltpu.sync_copy(zeros.at[pl.ds(s_ax * ROWS_PER_SUB, ROWS_PER_SUB)],
                    acc.at[pl.ds(s_ax * ROWS_PER_SUB, ROWS_PER_SUB)])
    plsc.subcore_barrier()

    def l_descs(c, b, r):
        base = (s_ax * CPS + c) * CH
        ds = [pltpu.make_async_copy(cols.at[pl.ds(base, CH)], cols_b[b], lsems[b]),
              pltpu.make_async_copy(vals.at[pl.ds(base, CH)], vals_b[b], lsems[b])]
        for j in range(NSUB):
            ds.append(pltpu.make_async_copy(
                rows.at[pl.ds(base + j * SUB, SUB)], rows_b[r].at[j], lsems[b]))
        return ds

    def g_descs(c, b):
        return [pltpu.make_async_copy(
                    x2.at[cols_b[b].at[pl.ds(j * SUB, SUB)]],
                    msgs_b[b].at[pl.ds(j * SUB, SUB)], gsems[b])
                for j in range(NSUB)]

    coff = jnp.full((16,), c_ax, dtype=jnp.int32)

    def adjust(b):
        cb = cols_b[b]

        def body(k, carry):
            v = cb[pl.ds(k * 16, 16)]
            cb[pl.ds(k * 16, 16)] = v + v + coff
            return carry
        lax.fori_loop(0, CH // 16, body, None, unroll=4)

    def w_start(c, b, r):
        for j in range(NSUB):
            pltpu.async_copy(msgs_b[b].at[pl.ds(j * SUB, SUB)],
                             acc.at[rows_b[r].at[j]], ssems[b], add=True)

    def w_wait(c, b, r):
        for j in range(NSUB):
            pltpu.make_async_copy(msgs_b[b].at[pl.ds(j * SUB, SUB)],
                                  acc.at[rows_b[r].at[j]], ssems[b]).wait()

    def scale(b):
        mb = msgs_b[b]
        vb = vals_b[b]

        def body(G, carry):
            vv = vb[pl.ds(G * 16, 16)]
            for t in range(16):
                bv = lax.broadcast(vv[t], (16,))
                e = G * 16 + t
                mb[e, pl.ds(0, 16)] = mb[e, pl.ds(0, 16)] * bv
                mb[e, pl.ds(16, 16)] = mb[e, pl.ds(16, 16)] * bv
            return carry
        lax.fori_loop(0, CH // 16, body, None)

    def chunk_step(c, q):
        b = q % 2
        r = q % 4

        @pl.when(c >= 2)
        def _():
            w_wait(c - 2, b, r)
        for d in g_descs(c, b):
            d.start()

        @pl.when(c >= 1)
        def _():
            for d in g_descs(c - 1, 1 - b):
                d.wait()
            scale(1 - b)
            w_start(c - 1, 1 - b, (q + 3) % 4)

        @pl.when(c + 1 < CPS)
        def _():
            nds = l_descs(c + 1, 1 - b, (q + 1) % 4)
            for d in nds:
                d.start()
            for d in nds:
                d.wait()
            adjust(1 - b)

    p = l_descs(0, 0, 0)
    for d in p:
        d.start()
    for d in p:
        d.wait()
    adjust(0)

    def quad(t, carry):
        for q in range(4):
            chunk_step(t * 4 + q, q)
        return carry
    lax.fori_loop(0, CPS // 4, quad, None)

    last = CPS - 1
    for d in g_descs(last, last % 2):
        d.wait()
    scale(last % 2)
    w_start(last, last % 2, last % 4)
    w_wait(last - 1, (last - 1) % 2, (last - 1) % 4)
    w_wait(last, last % 2, last % 4)

    plsc.subcore_barrier()

    pltpu.sync_copy(acc.at[pl.ds(s_ax * ROWS_PER_SUB, ROWS_PER_SUB)],
                    out.at[pl.ds(s_ax * ROWS_PER_SUB, ROWS_PER_SUB),
                           pl.ds(c_ax * F2, F2)])


_spmm = pl.kernel(
    _spmm_body,
    out_type=jax.ShapeDtypeStruct((NPAD, EMB), jnp.float32),
    mesh=plsc.VectorSubcoreMesh(core_axis_name="c", subcore_axis_name="s",
                                num_cores=NC, num_subcores=NS),
    compiler_params=pltpu.CompilerParams(use_tc_tiling_on_sc=False),
    scratch_types=[
        pltpu.VMEM_SHARED((NPAD, F2), jnp.float32),
        pltpu.VMEM((CH,), jnp.int32),
        pltpu.VMEM((CH,), jnp.int32),
        pltpu.VMEM((CH,), jnp.float32),
        pltpu.VMEM((CH,), jnp.float32),
        pltpu.VMEM((NSUB, SUB), jnp.int32),
        pltpu.VMEM((NSUB, SUB), jnp.int32),
        pltpu.VMEM((NSUB, SUB), jnp.int32),
        pltpu.VMEM((NSUB, SUB), jnp.int32),
        pltpu.VMEM((CH, F2), jnp.float32),
        pltpu.VMEM((CH, F2), jnp.float32),
        pltpu.SemaphoreType.DMA,
        pltpu.SemaphoreType.DMA,
        pltpu.SemaphoreType.DMA,
        pltpu.SemaphoreType.DMA,
        pltpu.SemaphoreType.DMA,
        pltpu.SemaphoreType.DMA,
    ],
)


def _gram_body(x0_ref, xk_ref, uH_ref, iH_ref, V_ref, m_ref, acc_ref):
    h = pl.program_id(0)
    j = pl.program_id(1)
    part = lax.dot_general(x0_ref[...], xk_ref[...],
                           (((0,), (0,)), ((), ())),
                           preferred_element_type=jnp.float32)

    @pl.when(j == 0)
    def _():
        acc_ref[...] = part

    @pl.when(j > 0)
    def _():
        acc_ref[...] = acc_ref[...] + part

    @pl.when(j == NBLK_H - 1)
    def _():
        W = jnp.where(h == 0, uH_ref[...], iH_ref[...])
        lat = _leaky(lax.dot_general(W, acc_ref[...],
                                     (((0,), (0,)), ((), ())),
                                     preferred_element_type=jnp.float32))
        for _ in range(C_LAYERS):
            lat = _leaky(jnp.dot(V_ref[...], lat,
                                 preferred_element_type=jnp.float32)) + lat
        m_ref[0] = jnp.dot(W, lat, preferred_element_type=jnp.float32)


def _gram(x0, xk, uHyper, iHyper, V):
    return pl.pallas_call(
        _gram_body,
        grid=(2, NBLK_H),
        in_specs=[
            pl.BlockSpec((BLK, EMB), lambda h, j: (h * NBLK_H + j, 0)),
            pl.BlockSpec((BLK, EMB), lambda h, j: (h * NBLK_H + j, 0)),
            pl.BlockSpec((EMB, HYP), lambda h, j: (0, 0)),
            pl.BlockSpec((EMB, HYP), lambda h, j: (0, 0)),
            pl.BlockSpec((HYP, HYP), lambda h, j: (0, 0)),
        ],
        out_specs=pl.BlockSpec((1, EMB, EMB), lambda h, j: (h, 0, 0)),
        out_shape=jax.ShapeDtypeStruct((2, EMB, EMB), jnp.float32),
        scratch_shapes=[pltpu.VMEM((EMB, EMB), jnp.float32)],
    )(x0, xk, uHyper, iHyper, V)


def _combine0_body(s_ref, x0_ref, m_ref, gnn_ref, hyp_ref, xn_ref):
    g = _leaky(s_ref[...])
    hypb = _leaky(jnp.dot(x0_ref[...], m_ref[0],
                          preferred_element_type=jnp.float32))
    gnn_ref[...] = g
    hyp_ref[...] = hypb
    xn_ref[...] = g + hypb


def _combine0(s, x0, m):
    return pl.pallas_call(
        _combine0_body,
        grid=(NBLK,),
        in_specs=[
            pl.BlockSpec((BLK, EMB), lambda i: (i, 0)),
            pl.BlockSpec((BLK, EMB), lambda i: (i, 0)),
            pl.BlockSpec((1, EMB, EMB), lambda i: (i // NBLK_H, 0, 0)),
        ],
        out_specs=[
            pl.BlockSpec((BLK, EMB), lambda i: (i, 0)),
            pl.BlockSpec((BLK, EMB), lambda i: (i, 0)),
            pl.BlockSpec((BLK, EMB), lambda i: (i, 0)),
        ],
        out_shape=[
            jax.ShapeDtypeStruct((N, EMB), jnp.float32),
            jax.ShapeDtypeStruct((N, EMB), jnp.float32),
            jax.ShapeDtypeStruct((N, EMB), jnp.float32),
        ],
    )(s, x0, m)


def _combine1_body(s_ref, x0_ref, x1_ref, m_ref, gnn_ref, hyp_ref, out_ref):
    g = _leaky(s_ref[...])
    hypb = _leaky(jnp.dot(x0_ref[...], m_ref[0],
                          preferred_element_type=jnp.float32))
    gnn_ref[...] = g
    hyp_ref[...] = hypb
    out_ref[...] = x0_ref[...] + x1_ref[...] + g + hypb


def _combine1(s, x0, x1, m):
    return pl.pallas_call(
        _combine1_body,
        grid=(NBLK,),
        in_specs=[
            pl.BlockSpec((BLK, EMB), lambda i: (i, 0)),
            pl.BlockSpec((BLK, EMB), lambda i: (i, 0)),
            pl.BlockSpec((BLK, EMB), lambda i: (i, 0)),
            pl.BlockSpec((1, EMB, EMB), lambda i: (i // NBLK_H, 0, 0)),
        ],
        out_specs=[
            pl.BlockSpec((BLK, EMB), lambda i: (i, 0)),
            pl.BlockSpec((BLK, EMB), lambda i: (i, 0)),
            pl.BlockSpec((BLK, EMB), lambda i: (i, 0)),
        ],
        out_shape=[
            jax.ShapeDtypeStruct((N, EMB), jnp.float32),
            jax.ShapeDtypeStruct((N, EMB), jnp.float32),
            jax.ShapeDtypeStruct((N, EMB), jnp.float32),
        ],
    )(s, x0, x1, m)


_npe = E_PAD - E
_ar = np.arange(_npe)
_ROWS_TAIL = np.asarray(N + _ar % (NPAD - N), dtype=np.int32)
_COLS_TAIL = np.asarray(_ar % N, dtype=np.int32)
_VALS_TAIL = np.zeros((_npe,), dtype=np.float32)


def kernel(adj_indices, adj_values, uEmbeds, iEmbeds, uHyper, iHyper, V, keepRate):
    x0 = jnp.concatenate([uEmbeds, iEmbeds], axis=0)

    rows = jnp.concatenate([adj_indices[0], jnp.asarray(_ROWS_TAIL)])
    cols = jnp.concatenate([adj_indices[1], jnp.asarray(_COLS_TAIL)])
    vals = jnp.concatenate([adj_values, jnp.asarray(_VALS_TAIL)])
    zeros = jnp.zeros((NPAD, F2), jnp.float32)

    s0 = _spmm(x0.reshape(2 * N, F2), rows, cols, vals, zeros)
    m0 = _gram(x0, x0, uHyper, iHyper, V)
    gnn0, hyp0, x1 = _combine0(s0, x0, m0)

    s1 = _spmm(x1.reshape(2 * N, F2), rows, cols, vals, zeros)
    m1 = _gram(x0, x1, uHyper, iHyper, V)
    gnn1, hyp1, out = _combine1(s1, x0, x1, m1)

    return (out, (gnn0, gnn1), (hyp0, hyp1))
